# baseline (speedup 1.0000x reference)
import jax
import jax.numpy as jnp
from jax import lax
from jax.experimental import pallas as pl
from jax.experimental.pallas import tpu as pltpu
from jax.experimental.pallas import tpu_sc as plsc

_B, _S, _D, _T = 4, 4096, 2048, 64
_N = _T + _S
_NC, _NS = 2, 16
_W = _NC * _NS
_P = _W // _B
_R = 8
_NT = _S // _P // _R
_NBUF = 4
_NOUTER = _NT // _NBUF


def _al(x):
    return pl.multiple_of(x, _R)


def _sc_body(text, task, mask, labels, oe, om, ol,
             buf, tbuf, mbuf, lbuf, pbuf_m, pbuf_l,
             isem, osem, tsem, tosem):
    w = lax.axis_index("c") * _NS + lax.axis_index("s")
    b = w // _P
    p = w % _P
    src0 = p * (_NT * _R)
    dst0 = _T + src0

    pltpu.async_copy(task.at[b, pl.ds(_al(p * _R), _R), :], tbuf, tsem)

    for u in range(_NBUF):
        pltpu.async_copy(text.at[b, pl.ds(_al(src0 + u * _R), _R), :],
                         buf.at[u], isem.at[u])

    pltpu.make_async_copy(task.at[b, pl.ds(_al(p * _R), _R), :],
                          tbuf, tsem).wait()
    pltpu.async_copy(tbuf, oe.at[b, pl.ds(_al(p * _R), _R), :], tosem)

    @pl.when(p == 0)
    def _mask_labels():
        for q in range(_T // 16):
            pbuf_m[pl.ds(q * 16, 16)] = jnp.ones((16,), jnp.int32)
            pbuf_l[pl.ds(q * 16, 16)] = jnp.full((16,), -100, jnp.int32)
        mrow = pl.multiple_of(b * _S, 8)
        orow = pl.multiple_of(b * _N, 8)
        orow_t = pl.multiple_of(b * _N + _T, 8)
        pltpu.sync_copy(pbuf_m, om.at[pl.ds(orow, _T)])
        pltpu.sync_copy(pbuf_l, ol.at[pl.ds(orow, _T)])
        pltpu.sync_copy(mask.at[pl.ds(mrow, _S)], mbuf)
        pltpu.sync_copy(mbuf, om.at[pl.ds(orow_t, _S)])
        pltpu.sync_copy(labels.at[pl.ds(mrow, _S)], lbuf)
        pltpu.sync_copy(lbuf, ol.at[pl.ds(orow_t, _S)])

    def outer(g, carry):
        for u in range(_NBUF):
            j = g * _NBUF + u
            pltpu.make_async_copy(text.at[b, pl.ds(_al(src0 + j * _R), _R), :],
                                  buf.at[u], isem.at[u]).wait()
            pltpu.async_copy(buf.at[u],
                             oe.at[b, pl.ds(_al(dst0 + j * _R), _R), :],
                             osem.at[u])
        for u in range(_NBUF):
            jn = (g + 1) * _NBUF + u

            @pl.when(jn < _NT)
            def _prefetch():
                pltpu.make_async_copy(
                    buf.at[u],
                    oe.at[b, pl.ds(_al(dst0 + (jn - _NBUF) * _R), _R), :],
                    osem.at[u]).wait()
                pltpu.async_copy(text.at[b, pl.ds(_al(src0 + jn * _R), _R), :],
                                 buf.at[u], isem.at[u])
        return carry

    lax.fori_loop(0, _NOUTER, outer, 0)

    for u in range(_NBUF):
        j = (_NOUTER - 1) * _NBUF + u
        pltpu.make_async_copy(buf.at[u],
                              oe.at[b, pl.ds(_al(dst0 + j * _R), _R), :],
                              osem.at[u]).wait()
    pltpu.make_async_copy(tbuf, oe.at[b, pl.ds(_al(p * _R), _R), :],
                          tosem).wait()


@jax.jit
def _inject(text_embeds, attention_mask, labels, task_embeds):
    mesh = plsc.VectorSubcoreMesh(core_axis_name="c", subcore_axis_name="s",
                                  num_cores=_NC, num_subcores=_NS)
    oe, om, ol = pl.kernel(
        _sc_body,
        out_type=(
            jax.ShapeDtypeStruct((_B, _N, _D), jnp.float32),
            jax.ShapeDtypeStruct((_B * _N,), jnp.int32),
            jax.ShapeDtypeStruct((_B * _N,), jnp.int32),
        ),
        mesh=mesh,
        scratch_types=(
            pltpu.VMEM((_NBUF, _R, _D), jnp.float32),
            pltpu.VMEM((_R, _D), jnp.float32),
            pltpu.VMEM((_S,), jnp.int32),
            pltpu.VMEM((_S,), jnp.int32),
            pltpu.VMEM((_T,), jnp.int32),
            pltpu.VMEM((_T,), jnp.int32),
            pltpu.SemaphoreType.DMA((_NBUF,)),
            pltpu.SemaphoreType.DMA((_NBUF,)),
            pltpu.SemaphoreType.DMA,
            pltpu.SemaphoreType.DMA,
        ),
    )(text_embeds, task_embeds,
      attention_mask.reshape(-1), labels.reshape(-1))
    return oe, om.reshape(_B, _N), ol.reshape(_B, _N)


def kernel(text_embeds, attention_mask, labels, task_embeds):
    b, s, d = text_embeds.shape
    t = task_embeds.shape[1]
    assert (b, s, d, t) == (_B, _S, _D, _T)
    return _inject(text_embeds, attention_mask, labels, task_embeds)


# speedup vs baseline: 1.3789x; 1.3789x over previous
import jax
import jax.numpy as jnp
from jax.experimental import pallas as pl
from jax.experimental.pallas import tpu as pltpu

_B, _S, _D, _T = 4, 4096, 2048, 64
_N = _T + _S
_CR = 256
_NCH = _S // _CR
_NBUF = 8
_LAG = 4


def _body(text, mask, labels, task, oe, om, ol, buf, isem, osem):
    chunks = []
    for b in range(_B):
        chunks.append((task, b, 0, b, 0, _T))
        for j in range(_NCH):
            chunks.append((text, b, j * _CR, b, _T + j * _CR, _CR))

    n = len(chunks)

    def in_copy(k):
        src, sb, sr, db, dr, rows = chunks[k]
        return pltpu.make_async_copy(
            src.at[sb, pl.ds(sr, rows), :],
            buf.at[k % _NBUF, pl.ds(0, rows), :], isem.at[k % _NBUF])

    def out_copy(k):
        src, sb, sr, db, dr, rows = chunks[k]
        return pltpu.make_async_copy(
            buf.at[k % _NBUF, pl.ds(0, rows), :],
            oe.at[db, pl.ds(dr, rows), :], osem.at[k % _NBUF])

    for k in range(n + _LAG):
        if k < n:
            if k >= _NBUF:
                out_copy(k - _NBUF).wait()
            in_copy(k).start()
        if k >= _LAG:
            j = k - _LAG
            in_copy(j).wait()
            out_copy(j).start()

    om[...] = jnp.concatenate(
        [jnp.ones((_B, _T), dtype=om.dtype), mask[...]], axis=1)
    ol[...] = jnp.concatenate(
        [jnp.full((_B, _T), -100, dtype=ol.dtype), labels[...]], axis=1)

    for k in range(n - _NBUF, n):
        out_copy(k).wait()


def kernel(text_embeds, attention_mask, labels, task_embeds):
    any_spec = pl.BlockSpec(memory_space=pl.ANY)
    vmem_spec = pl.BlockSpec(memory_space=pltpu.MemorySpace.VMEM)
    return pl.pallas_call(
        _body,
        in_specs=[any_spec, vmem_spec, vmem_spec, any_spec],
        out_specs=[any_spec, vmem_spec, vmem_spec],
        out_shape=(
            jax.ShapeDtypeStruct((_B, _N, _D), jnp.float32),
            jax.ShapeDtypeStruct((_B, _N), jnp.int32),
            jax.ShapeDtypeStruct((_B, _N), jnp.int32),
        ),
        scratch_shapes=[
            pltpu.VMEM((_NBUF, _CR, _D), jnp.float32),
            pltpu.SemaphoreType.DMA((_NBUF,)),
            pltpu.SemaphoreType.DMA((_NBUF,)),
        ],
    )(text_embeds, attention_mask, labels, task_embeds)
